# baseline (speedup 1.0000x reference)
import functools

import jax
import jax.numpy as jnp
from jax import lax
from jax.experimental import pallas as pl
from jax.experimental.pallas import tpu as pltpu
from jax.experimental.pallas import tpu_sc as plsc

_M = 10000
_B = 4096
_IMG = (3, 32, 32)
_J = 3 * 32 * 32

_NC = 2
_NS = 16
_NW = _NC * _NS
_PPW = _J // _NW
_L = 16
_BPW = _B // _NW


def _mask_body(keys_ref, mask_ref):
  pid = pl.program_id(0)
  kb = keys_ref[0, pl.ds(pid * _BPW, _BPW)].reshape(_BPW, 1)
  kall = keys_ref[...].reshape(1, _B)
  i_col = pid * _BPW + lax.broadcasted_iota(jnp.int32, (_BPW, 1), 0)
  j_row = lax.broadcasted_iota(jnp.int32, (1, _B), 1)
  dup = jnp.where((kb == kall) & (j_row > i_col), 1, 0)
  mask_ref[0, pl.ds(pid * _BPW, _BPW)] = 1 - jnp.max(dup, axis=1)


_mask_call = pl.pallas_call(
    _mask_body,
    grid=(_NW,),
    in_specs=[pl.BlockSpec((1, _B), lambda i: (0, 0))],
    out_specs=pl.BlockSpec((1, _B), lambda i: (0, 0)),
    out_shape=jax.ShapeDtypeStruct((1, _B), jnp.int32),
)


def _sc_body(mem_hbm, x_hbm, keys_hbm, vals_hbm, y_hbm, lab_hbm, mask_hbm,
             out_hbm, lab_out_hbm,
             keys_a, vals_a, mask_a, pb0, pb1, xb0, xb1, lab_all, y_all,
             sin0, sin1, sx0, sx1, sout0, sout1):
  wid = lax.axis_index("s") * _NC + lax.axis_index("c")
  base = wid * _PPW
  pb = (pb0, pb1)
  xb = (xb0, xb1)
  sin = (sin0, sin1)
  sx = (sx0, sx1)
  sout = (sout0, sout1)

  pltpu.sync_copy(keys_hbm, keys_a)
  pltpu.sync_copy(vals_hbm, vals_a)
  pltpu.sync_copy(mask_hbm, mask_a)

  @pl.when(wid == 0)
  def _():
    pltpu.sync_copy(y_hbm, y_all)
    pltpu.sync_copy(lab_hbm, lab_all)

    def lbody(t, c):
      kv = keys_a[pl.ds(t * _L, _L)]
      vv = vals_a[pl.ds(t * _L, _L)]
      mv = mask_a[pl.ds(t * _L, _L)] > 0
      yv = plsc.load_gather(y_all, [vv])
      plsc.store_scatter(lab_all, [kv], yv, mask=mv)
      return c

    lax.fori_loop(0, _B // _L, lbody, 0)
    pltpu.sync_copy(lab_all, lab_out_hbm)

  def gbody(g, carry):
    for b in range(2):
      j = base + g * 2 + b

      @pl.when(g > 0)
      def _(j=j, b=b):
        pltpu.make_async_copy(pb[b], out_hbm.at[j - 2], sout[b]).wait()

      pltpu.async_copy(mem_hbm.at[j], pb[b], sin[b])
      pltpu.async_copy(x_hbm.at[j], xb[b], sx[b])

    for b in range(2):
      j = base + g * 2 + b
      pltpu.make_async_copy(mem_hbm.at[j], pb[b], sin[b]).wait()
      pltpu.make_async_copy(x_hbm.at[j], xb[b], sx[b]).wait()

      def patch(t, c, b=b):
        kv = keys_a[pl.ds(t * _L, _L)]
        vv = vals_a[pl.ds(t * _L, _L)]
        mv = mask_a[pl.ds(t * _L, _L)] > 0
        xv = plsc.load_gather(xb[b], [vv])
        plsc.store_scatter(pb[b], [kv], xv, mask=mv)
        return c

      lax.fori_loop(0, _B // _L, patch, 0)
      pltpu.async_copy(pb[b], out_hbm.at[j], sout[b])

    return carry

  lax.fori_loop(0, _PPW // 2, gbody, 0)

  pltpu.make_async_copy(pb[0], out_hbm.at[base], sout[0]).wait()
  pltpu.make_async_copy(pb[1], out_hbm.at[base], sout[1]).wait()


@functools.cache
def _get_sc_call():
  return functools.partial(
      pl.kernel,
      out_type=(
          jax.ShapeDtypeStruct((_J, _M), jnp.float32),
          jax.ShapeDtypeStruct((_M,), jnp.int32),
      ),
      mesh=plsc.VectorSubcoreMesh(core_axis_name="c", subcore_axis_name="s"),
      compiler_params=pltpu.CompilerParams(needs_layout_passes=False),
      scratch_types=[
          pltpu.VMEM((_B,), jnp.int32),
          pltpu.VMEM((_B,), jnp.int32),
          pltpu.VMEM((_B,), jnp.int32),
          pltpu.VMEM((_M,), jnp.float32),
          pltpu.VMEM((_M,), jnp.float32),
          pltpu.VMEM((_B,), jnp.float32),
          pltpu.VMEM((_B,), jnp.float32),
          pltpu.VMEM((_M,), jnp.int32),
          pltpu.VMEM((_B,), jnp.int32),
          pltpu.SemaphoreType.DMA,
          pltpu.SemaphoreType.DMA,
          pltpu.SemaphoreType.DMA,
          pltpu.SemaphoreType.DMA,
          pltpu.SemaphoreType.DMA,
          pltpu.SemaphoreType.DMA,
      ],
  )(_sc_body)


def kernel(mem, buffer_label, idx_keys, idx_vals, x, y):
  mask = _mask_call(idx_keys.reshape(1, _B)).reshape(_B)
  mem_t = mem.transpose(1, 2, 3, 0).reshape(_J, _M)
  x_t = x.transpose(1, 2, 3, 0).reshape(_J, _B)
  out_t, out_lab = _get_sc_call()(mem_t, x_t, idx_keys, idx_vals, y,
                                  buffer_label, mask)
  out_mem = out_t.reshape(_IMG + (_M,)).transpose(3, 0, 1, 2)
  return out_mem, out_lab


# speedup vs baseline: 1.2979x; 1.2979x over previous
import functools

import jax
import jax.numpy as jnp
from jax import lax
from jax.experimental import pallas as pl
from jax.experimental.pallas import tpu as pltpu
from jax.experimental.pallas import tpu_sc as plsc

_M = 10000
_B = 4096
_IMG = (3, 32, 32)
_J = 3 * 32 * 32

_NC = 2
_NS = 16
_NW = _NC * _NS
_PPW = _J // _NW
_PB = 8
_NBLK = _PPW // _PB
_L = 16
_BPW = _B // _NW


def _mask_body(keys_ref, vals_ref, upd_ref):
  pid = pl.program_id(0)
  kb = keys_ref[0, pl.ds(pid * _BPW, _BPW)]
  kall = keys_ref[...].reshape(1, _B)
  i_col = pid * _BPW + lax.broadcasted_iota(jnp.int32, (_BPW, 1), 0)
  j_row = lax.broadcasted_iota(jnp.int32, (1, _B), 1)
  dup = jnp.where((kb.reshape(_BPW, 1) == kall) & (j_row > i_col), 1, 0)
  active = jnp.max(dup, axis=1) == 0
  vb = vals_ref[0, pl.ds(pid * _BPW, _BPW)]
  u = jnp.where(active, kb | (vb << 14), -1)
  upd_ref[0, pl.ds(pid * _BPW, _BPW)] = u


_mask_call = pl.pallas_call(
    _mask_body,
    grid=(_NW,),
    in_specs=[pl.BlockSpec((1, _B), lambda i: (0, 0))] * 2,
    out_specs=pl.BlockSpec((1, _B), lambda i: (0, 0)),
    out_shape=jax.ShapeDtypeStruct((1, _B), jnp.int32),
)


def _sc_body(mem_hbm, x_hbm, upd_hbm, labf_hbm, yf_hbm,
             out_hbm, labf_out_hbm,
             upd_a, pb, xb, sin, sx, sout):
  wid = lax.axis_index("s") * _NC + lax.axis_index("c")
  base = wid * _PPW

  pltpu.sync_copy(upd_hbm, upd_a)

  @pl.when(wid == 0)
  def _():
    pltpu.sync_copy(labf_hbm, pb.at[0])
    pltpu.sync_copy(yf_hbm, pb.at[1, pl.ds(0, _B)])

    def lbody(t, c):
      u = upd_a[pl.ds(t * _L, _L)]
      mv = u >= 0
      kv = u & 16383
      vv = lax.shift_right_logical(u, 14)
      yv = plsc.load_gather(pb, [jnp.full((_L,), 1, jnp.int32), _B * 0 + vv],
                            mask=mv)
      plsc.store_scatter(pb, [jnp.full((_L,), 0, jnp.int32), kv], yv, mask=mv)
      return c

    lax.fori_loop(0, _B // _L, lbody, 0)
    pltpu.sync_copy(pb.at[0], labf_out_hbm)

  def block(k, c):
    j8 = base + k * _PB

    @pl.when(k > 0)
    def _():
      pltpu.make_async_copy(pb, out_hbm.at[pl.ds(base, _PB)], sout).wait()

    cin = pltpu.async_copy(mem_hbm.at[pl.ds(j8, _PB)], pb, sin)
    cx = pltpu.async_copy(x_hbm.at[pl.ds(j8, _PB)], xb, sx)
    cin.wait()
    cx.wait()

    def patch(t, c2):
      u = upd_a[pl.ds(t * _L, _L)]
      mv = u >= 0
      kv = u & 16383
      vv = lax.shift_right_logical(u, 14)
      for p in range(_PB):
        pv = jnp.full((_L,), p, jnp.int32)
        xv = plsc.load_gather(xb, [pv, vv], mask=mv)
        plsc.store_scatter(pb, [pv, kv], xv, mask=mv)
      return c2

    lax.fori_loop(0, _B // _L, patch, 0)
    pltpu.async_copy(pb, out_hbm.at[pl.ds(j8, _PB)], sout)
    return c

  lax.fori_loop(0, _NBLK, block, 0)
  pltpu.make_async_copy(pb, out_hbm.at[pl.ds(base, _PB)], sout).wait()


@functools.cache
def _get_sc_call():
  return functools.partial(
      pl.kernel,
      out_type=(
          jax.ShapeDtypeStruct((_J, _M), jnp.float32),
          jax.ShapeDtypeStruct((_M,), jnp.float32),
      ),
      mesh=plsc.VectorSubcoreMesh(core_axis_name="c", subcore_axis_name="s"),
      compiler_params=pltpu.CompilerParams(needs_layout_passes=False),
      scratch_types=[
          pltpu.VMEM((_B,), jnp.int32),
          pltpu.VMEM((_PB, _M), jnp.float32),
          pltpu.VMEM((_PB, _B), jnp.float32),
          pltpu.SemaphoreType.DMA,
          pltpu.SemaphoreType.DMA,
          pltpu.SemaphoreType.DMA,
      ],
  )(_sc_body)


def kernel(mem, buffer_label, idx_keys, idx_vals, x, y):
  upd = _mask_call(idx_keys.reshape(1, _B), idx_vals.reshape(1, _B))
  mem_t = mem.transpose(1, 2, 3, 0).reshape(_J, _M)
  x_t = x.transpose(1, 2, 3, 0).reshape(_J, _B)
  labf = lax.bitcast_convert_type(buffer_label, jnp.float32)
  yf = lax.bitcast_convert_type(y, jnp.float32)
  out_t, out_labf = _get_sc_call()(mem_t, x_t, upd.reshape(_B), labf, yf)
  out_mem = out_t.reshape(_IMG + (_M,)).transpose(3, 0, 1, 2)
  return out_mem, lax.bitcast_convert_type(out_labf, jnp.int32)
